# baseline (speedup 1.0000x reference)
import functools

import jax
import jax.numpy as jnp
from jax import lax
from jax.experimental import pallas as pl
from jax.experimental.pallas import tpu as pltpu
from jax.experimental.pallas import tpu_sc as plsc

_SIZES = (512, 4096, 4096, 256)
_DEG = 16
_B = 256
_L = 16
_NBLK = _B // _L
_PMAX = 4096
_CDMAX = 512
_NBUF = 2


def _fused():
    mesh = plsc.VectorSubcoreMesh(core_axis_name="c", subcore_axis_name="s")

    scratch = [pltpu.VMEM((_L, _PMAX), jnp.float32)]
    for _ in range(_NBUF):
        scratch += [
            pltpu.VMEM((_DEG, _CDMAX), jnp.int32),
            pltpu.VMEM((_DEG, _CDMAX), jnp.float32),
            pltpu.VMEM((_CDMAX,), jnp.float32),
            pltpu.VMEM((_L, _CDMAX), jnp.float32),
        ]
    scratch += [pltpu.SemaphoreType.DMA] * (1 + 2 * _NBUF)

    @functools.partial(
        pl.kernel,
        out_type=(jax.ShapeDtypeStruct((_NBLK, _L, _SIZES[3]), jnp.float32),
                  jax.ShapeDtypeStruct((_NBLK, _L, _SIZES[1]), jnp.float32),
                  jax.ShapeDtypeStruct((_NBLK, _L, _SIZES[2]), jnp.float32)),
        mesh=mesh,
        compiler_params=pltpu.CompilerParams(needs_layout_passes=False),
        scratch_types=scratch,
    )
    def fused_k(x_hbm, s0_hbm, w0_hbm, b0_hbm, s1_hbm, w1_hbm, b1_hbm,
                s2_hbm, w2_hbm, b2_hbm, out_hbm, act_a, act_b, table_v, *rest):
        bufs = [rest[4 * i:4 * i + 4] for i in range(_NBUF)]
        sem_t = rest[4 * _NBUF]
        sem_in = rest[4 * _NBUF + 1:4 * _NBUF + 1 + _NBUF]
        sem_out = rest[4 * _NBUF + 1 + _NBUF:]

        s = lax.axis_index("s")
        c = lax.axis_index("c")
        blk = c * 8 + s // 2
        sblk = s // 2
        half = s % 2

        def run_layer(prev, curr, table_src, src_hbm, w_hbm, b_hbm, out_dst):
            cph = curr // 2
            cd = min(cph, _CDMAX)
            nch = cph // cd
            ng = cd // _L
            d_of = lambda ci: half * cph + ci * cd

            tbl_cp = pltpu.async_copy(
                table_src, table_v.at[:, pl.ds(0, prev)], sem_t)

            def start_inputs(ci):
                buf = ci % _NBUF
                ib, wb, bb, _ = bufs[buf]
                d0 = d_of(ci)
                return (
                    pltpu.async_copy(src_hbm.at[:, pl.ds(d0, cd)],
                                     ib.at[:, pl.ds(0, cd)], sem_in[buf]),
                    pltpu.async_copy(w_hbm.at[:, pl.ds(d0, cd)],
                                     wb.at[:, pl.ds(0, cd)], sem_in[buf]),
                    pltpu.async_copy(b_hbm.at[pl.ds(d0, cd)],
                                     bb.at[pl.ds(0, cd)], sem_in[buf]),
                )

            pending_in = start_inputs(0)
            tbl_cp.wait()
            pending_out = [None] * _NBUF
            for ci in range(nch):
                buf = ci % _NBUF
                for h in pending_in:
                    h.wait()
                if ci + 1 < nch:
                    pending_in = start_inputs(ci + 1)
                ib, wb, bb, ob = bufs[buf]
                if pending_out[buf] is not None:
                    pending_out[buf].wait()

                def group(g, _, ib=ib, wb=wb, bb=bb, ob=ob):
                    col0 = pl.multiple_of(g * _L, _L)
                    bias_vec = bb[pl.ds(col0, _L)]
                    accs = [bias_vec] * _L
                    for k in range(_DEG):
                        icol = ib[k, pl.ds(col0, _L)]
                        wcol = wb[k, pl.ds(col0, _L)]
                        for j in range(_L):
                            jvec = jnp.full((_L,), j, jnp.int32)
                            accs[j] = accs[j] + wcol * plsc.load_gather(
                                table_v, [jvec, icol])
                    for j in range(_L):
                        ob[j, pl.ds(col0, _L)] = jnp.maximum(accs[j], 0.0)
                    return 0

                lax.fori_loop(0, ng, group, 0)
                pending_out[buf] = pltpu.async_copy(
                    ob.at[:, pl.ds(0, cd)], out_dst(d_of(ci), cd), sem_out[buf])
            for h in pending_out:
                if h is not None:
                    h.wait()

        run_layer(_SIZES[0], _SIZES[1], x_hbm.at[blk], s0_hbm, w0_hbm, b0_hbm,
                  lambda d0, cd: act_a.at[blk, :, pl.ds(d0, cd)])
        plsc.subcore_barrier()
        run_layer(_SIZES[1], _SIZES[2], act_a.at[blk], s1_hbm, w1_hbm, b1_hbm,
                  lambda d0, cd: act_b.at[blk, :, pl.ds(d0, cd)])
        plsc.subcore_barrier()
        run_layer(_SIZES[2], _SIZES[3], act_b.at[blk], s2_hbm, w2_hbm, b2_hbm,
                  lambda d0, cd: out_hbm.at[blk, :, pl.ds(d0, cd)])

    return fused_k


_FUSED = _fused()


def kernel(x, edge_index_0, weights_0, bias_0, edge_index_1, weights_1, bias_1,
           edge_index_2, weights_2, bias_2):
    xb = x.reshape(_NBLK, _L, _SIZES[0])
    cols = []
    for li, (ei, w) in enumerate(((edge_index_0, weights_0),
                                  (edge_index_1, weights_1),
                                  (edge_index_2, weights_2))):
        curr = _SIZES[li + 1]
        cols.append(ei[1].astype(jnp.int32).reshape(curr, _DEG).T)
        cols.append(w.reshape(curr, _DEG).T)
    out, _, _ = _FUSED(xb, cols[0], cols[1], bias_0, cols[2], cols[3], bias_1,
                       cols[4], cols[5], bias_2)
    return out.reshape(_B, _SIZES[3])
